# baseline (speedup 1.0000x reference)
import functools

import jax
import jax.numpy as jnp
import numpy as np
from jax import lax
from jax.experimental import pallas as pl
from jax.experimental.pallas import tpu as pltpu
from jax.experimental.pallas import tpu_sc as plsc

N = 10000
E = 320000
C = 128
NC = 2
NS = 16
NW = NC * NS
EPW = E // NW
K = 80
NCH = EPW // K
NBUF = 2
RPT = N // NS

_PERM_NP = np.zeros((C, C), np.float32)
for _g in range(C // 32):
    for _i in range(16):
        _PERM_NP[_g * 32 + _i, _g * 32 + 2 * _i] = 1.0
        _PERM_NP[_g * 32 + 16 + _i, _g * 32 + 2 * _i + 1] = 1.0


def _vsc_mesh():
    return plsc.VectorSubcoreMesh(core_axis_name="c", subcore_axis_name="s")


@functools.partial(
    pl.kernel,
    out_type=jax.ShapeDtypeStruct((NC, 2, N), jnp.float32),
    mesh=_vsc_mesh(),
    scratch_types=[
        pltpu.VMEM((EPW,), jnp.int32),
        pltpu.VMEM((EPW,), jnp.int32),
        pltpu.VMEM((K,), jnp.float32),
        pltpu.VMEM((2000,), jnp.float32),
        pltpu.VMEM_SHARED((N,), jnp.float32),
        pltpu.VMEM_SHARED((N,), jnp.float32),
        pltpu.SemaphoreType.DMA,
    ],
    compiler_params=pltpu.CompilerParams(use_tc_tiling_on_sc=False),
)
def _deg(src_hbm, dst_hbm, degp_hbm, sidx, didx, ones, zbuf, acc_s, acc_d,
         dsem):
    cid = lax.axis_index("c")
    sid = lax.axis_index("s")
    wid = cid * NS + sid

    @pl.when(sid == 0)
    def _init():
        def zrow(i, carry):
            zbuf[pl.ds(i * 16, 16)] = jnp.zeros((16,), jnp.float32)
            return carry
        lax.fori_loop(0, 2000 // 16, zrow, None)
        for t in range(N // 2000):
            pltpu.sync_copy(zbuf, acc_s.at[pl.ds(t * 2000, 2000)])
            pltpu.sync_copy(zbuf, acc_d.at[pl.ds(t * 2000, 2000)])

    for c16 in range(K // 16):
        ones[pl.ds(c16 * 16, 16)] = jnp.ones((16,), jnp.float32)

    plsc.subcore_barrier()

    base = pl.multiple_of(wid * EPW, 8)
    pltpu.sync_copy(src_hbm.at[pl.ds(base, EPW)], sidx)
    pltpu.sync_copy(dst_hbm.at[pl.ds(base, EPW)], didx)

    DB = 5

    def _cidx(ref, j):
        return ref.at[pl.ds(pl.multiple_of(j * K, 8), K)]

    def _fire(b0):
        for b in range(DB):
            pltpu.async_copy(ones, acc_s.at[_cidx(sidx, b0 + b)], dsem,
                             add=True)
            pltpu.async_copy(ones, acc_d.at[_cidx(didx, b0 + b)], dsem,
                             add=True)

    def _drain():
        for _ in range(2 * DB):
            pltpu.make_async_copy(ones, acc_s.at[_cidx(sidx, 0)],
                                  dsem).wait()

    _fire(0)

    def chunk(o, carry):
        _fire((o + 1) * DB)
        _drain()
        return carry
    lax.fori_loop(0, NCH // DB - 1, chunk, None)
    _drain()

    plsc.subcore_barrier()

    @pl.when(sid == 0)
    def _writeout():
        pltpu.sync_copy(acc_s, degp_hbm.at[cid, 0])
        pltpu.sync_copy(acc_d, degp_hbm.at[cid, 1])


@functools.partial(
    pl.kernel,
    out_type=jax.ShapeDtypeStruct((NC, N, C), jnp.float32),
    mesh=_vsc_mesh(),
    scratch_types=[
        pltpu.VMEM((EPW,), jnp.int32),
        pltpu.VMEM((EPW,), jnp.int32),
        [pltpu.VMEM((K, C), jnp.bfloat16) for _ in range(NBUF)],
        [pltpu.VMEM((K, C), jnp.float32) for _ in range(NBUF)],
        [pltpu.SemaphoreType.DMA for _ in range(NBUF)],
        pltpu.VMEM_SHARED((N, C), jnp.float32),
    ],
    compiler_params=pltpu.CompilerParams(use_tc_tiling_on_sc=False,
                                         needs_layout_passes=False),
)
def _agg(src_hbm, dst_hbm, h_hbm, out_hbm, sidx, didx, gbuf, rows, sems, acc):
    cid = lax.axis_index("c")
    sid = lax.axis_index("s")
    wid = cid * NS + sid

    def zrow(i, carry):
        for c16 in range(C // 16):
            rows[0][i, pl.ds(c16 * 16, 16)] = jnp.zeros((16,), jnp.float32)
        return carry
    lax.fori_loop(0, K, zrow, None)
    for t in range(RPT // K):
        pltpu.sync_copy(rows[0], acc.at[pl.ds(sid * RPT + t * K, K)])
    _REM = RPT - (RPT // K) * K
    if _REM:
        pltpu.sync_copy(rows[0].at[pl.ds(0, _REM)],
                        acc.at[pl.ds(sid * RPT + (RPT // K) * K, _REM)])
    plsc.subcore_barrier()

    base = pl.multiple_of(wid * EPW, 8)
    pltpu.sync_copy(src_hbm.at[pl.ds(base, EPW)], sidx)
    pltpu.sync_copy(dst_hbm.at[pl.ds(base, EPW)], didx)

    def _cidx(ref, j):
        return ref.at[pl.ds(pl.multiple_of(j * K, 8), K)]

    def _gather(j, b):
        pltpu.async_copy(h_hbm.at[_cidx(sidx, j)], gbuf[b], sems[b])

    def _gwait(b):
        pltpu.make_async_copy(h_hbm.at[pl.ds(0, K)], gbuf[b], sems[b]).wait()

    MASK = jnp.int32(-65536)

    def _convert(b):
        def crow(r, carry):
            for c4 in range(C // 32):
                g = gbuf[b][r, pl.ds(c4 * 32, 32)]
                vi = plsc.bitcast(g, jnp.int32)
                lo = plsc.bitcast(vi << 16, jnp.float32)
                hi = plsc.bitcast(vi & MASK, jnp.float32)
                rows[b][r, pl.ds(c4 * 32, 16)] = lo
                rows[b][r, pl.ds(c4 * 32 + 16, 16)] = hi
            return carry
        lax.fori_loop(0, K, crow, None)

    def _scat(j, b):
        pltpu.sync_copy(rows[b], acc.at[_cidx(didx, j)], add=True)

    OUT = NCH // NBUF - 1
    TAIL = NCH - NBUF * OUT

    for b in range(NBUF):
        _gather(b, b)

    def outer(o, carry):
        jb = o * NBUF
        for b in range(NBUF):
            _gwait(b)
            _convert(b)
            _gather(jb + b + NBUF, b)
            _scat(jb + b, b)
        return carry
    lax.fori_loop(0, OUT, outer, None)

    jb = OUT * NBUF
    for b in range(NBUF):
        _gwait(b)
        _convert(b)
        _scat(jb + b, b)
    for t in range(TAIL - NBUF):
        j = jb + NBUF + t
        _gather(j, t)
        _gwait(t)
        _convert(t)
        _scat(j, t)

    plsc.subcore_barrier()
    pltpu.sync_copy(acc.at[pl.ds(sid * RPT, RPT)],
                    out_hbm.at[cid, pl.ds(sid * RPT, RPT)])


BM = 2048


def _norm_cols(degp_ref):
    deg = degp_ref[0] + degp_ref[1]
    nrm = lax.rsqrt(jnp.maximum(deg, 1.0))
    nt = jnp.transpose(nrm, (1, 0))
    return nt[:, 0:1], nt[:, 1:2]


def _prescale_body(x_ref, degp_ref, p_ref, h_ref):
    ns, _ = _norm_cols(degp_ref)
    hs = jnp.dot(x_ref[...] * ns, p_ref[...],
                 preferred_element_type=jnp.float32)
    h_ref[...] = hs.astype(jnp.bfloat16)


_prescale = pl.pallas_call(
    _prescale_body,
    grid=((N + BM - 1) // BM,),
    in_specs=[
        pl.BlockSpec((BM, C), lambda i: (i, 0)),
        pl.BlockSpec((NC, 2, BM), lambda i: (0, 0, i)),
        pl.BlockSpec((C, C), lambda i: (0, 0)),
    ],
    out_specs=pl.BlockSpec((BM, C), lambda i: (i, 0)),
    out_shape=jax.ShapeDtypeStruct((N, C), jnp.bfloat16),
)


def _mm1_body(aggp_ref, degp_ref, w_ref, p_ref, o_ref):
    ns, nd = _norm_cols(degp_ref)
    a = (aggp_ref[0] + aggp_ref[1]) * nd
    wp = jnp.dot(w_ref[...], p_ref[...], preferred_element_type=jnp.float32)
    h = jnp.dot(a, wp, preferred_element_type=jnp.float32)
    o_ref[...] = (jnp.maximum(h, 0.0) * ns).astype(jnp.bfloat16)


_mm1 = pl.pallas_call(
    _mm1_body,
    grid=((N + BM - 1) // BM,),
    in_specs=[
        pl.BlockSpec((NC, BM, C), lambda i: (0, i, 0)),
        pl.BlockSpec((NC, 2, BM), lambda i: (0, 0, i)),
        pl.BlockSpec((C, C), lambda i: (0, 0)),
        pl.BlockSpec((C, C), lambda i: (0, 0)),
    ],
    out_specs=pl.BlockSpec((BM, C), lambda i: (i, 0)),
    out_shape=jax.ShapeDtypeStruct((N, C), jnp.bfloat16),
)


def _mm2_body(aggp_ref, degp_ref, w_ref, o_ref):
    _, nd = _norm_cols(degp_ref)
    a = (aggp_ref[0] + aggp_ref[1]) * nd
    o_ref[...] = jnp.dot(a, w_ref[...], preferred_element_type=jnp.float32)


_mm2 = pl.pallas_call(
    _mm2_body,
    grid=((N + BM - 1) // BM,),
    in_specs=[
        pl.BlockSpec((NC, BM, C), lambda i: (0, i, 0)),
        pl.BlockSpec((NC, 2, BM), lambda i: (0, 0, i)),
        pl.BlockSpec((C, C), lambda i: (0, 0)),
    ],
    out_specs=pl.BlockSpec((BM, C), lambda i: (i, 0)),
    out_shape=jax.ShapeDtypeStruct((N, C), jnp.float32),
)


def kernel(x, edge_index, W1, W2):
    ei = edge_index.astype(jnp.int32)
    src = ei[0]
    dst = ei[1]

    perm = jnp.asarray(_PERM_NP)

    degp = _deg(src, dst)
    h = _prescale(x, degp, perm)
    aggp = _agg(src, dst, h)
    h1 = _mm1(aggp, degp, W1, perm)
    aggp2 = _agg(src, dst, h1)
    out = _mm2(aggp2, degp, W2)
    return out


# speedup vs baseline: 2.2652x; 2.2652x over previous
import functools

import jax
import jax.numpy as jnp
from jax import lax
from jax.experimental import pallas as pl
from jax.experimental.pallas import tpu as pltpu
from jax.experimental.pallas import tpu_sc as plsc

N = 10000
E = 320000
C = 128
NC = 2
NS = 16
NW = NC * NS
EPW = E // NW
K = 80
NCH = EPW // K
NBUF = 3
RPT = N // NS


def _vsc_mesh():
    return plsc.VectorSubcoreMesh(core_axis_name="c", subcore_axis_name="s")


@functools.partial(
    pl.kernel,
    out_type=jax.ShapeDtypeStruct((NC, 2, N), jnp.float32),
    mesh=_vsc_mesh(),
    scratch_types=[
        pltpu.VMEM((EPW,), jnp.int32),
        pltpu.VMEM((EPW,), jnp.int32),
        pltpu.VMEM((K,), jnp.float32),
        pltpu.VMEM((2000,), jnp.float32),
        pltpu.VMEM_SHARED((N,), jnp.float32),
        pltpu.VMEM_SHARED((N,), jnp.float32),
        pltpu.SemaphoreType.DMA,
    ],
    compiler_params=pltpu.CompilerParams(use_tc_tiling_on_sc=False),
)
def _deg(src_hbm, dst_hbm, degp_hbm, sidx, didx, ones, zbuf, acc_s, acc_d,
         dsem):
    cid = lax.axis_index("c")
    sid = lax.axis_index("s")
    wid = cid * NS + sid

    @pl.when(sid == 0)
    def _init():
        def zrow(i, carry):
            zbuf[pl.ds(i * 16, 16)] = jnp.zeros((16,), jnp.float32)
            return carry
        lax.fori_loop(0, 2000 // 16, zrow, None)
        for t in range(N // 2000):
            pltpu.sync_copy(zbuf, acc_s.at[pl.ds(t * 2000, 2000)])
            pltpu.sync_copy(zbuf, acc_d.at[pl.ds(t * 2000, 2000)])

    for c16 in range(K // 16):
        ones[pl.ds(c16 * 16, 16)] = jnp.ones((16,), jnp.float32)

    plsc.subcore_barrier()

    base = pl.multiple_of(wid * EPW, 8)
    pltpu.sync_copy(src_hbm.at[pl.ds(base, EPW)], sidx)
    pltpu.sync_copy(dst_hbm.at[pl.ds(base, EPW)], didx)

    DB = 5

    def _cidx(ref, j):
        return ref.at[pl.ds(pl.multiple_of(j * K, 8), K)]

    def _fire(b0):
        for b in range(DB):
            pltpu.async_copy(ones, acc_s.at[_cidx(sidx, b0 + b)], dsem,
                             add=True)
            pltpu.async_copy(ones, acc_d.at[_cidx(didx, b0 + b)], dsem,
                             add=True)

    def _drain():
        for _ in range(2 * DB):
            pltpu.make_async_copy(ones, acc_s.at[_cidx(sidx, 0)],
                                  dsem).wait()

    _fire(0)

    def chunk(o, carry):
        _fire((o + 1) * DB)
        _drain()
        return carry
    lax.fori_loop(0, NCH // DB - 1, chunk, None)
    _drain()

    plsc.subcore_barrier()

    @pl.when(sid == 0)
    def _writeout():
        pltpu.sync_copy(acc_s, degp_hbm.at[cid, 0])
        pltpu.sync_copy(acc_d, degp_hbm.at[cid, 1])


@functools.partial(
    pl.kernel,
    out_type=jax.ShapeDtypeStruct((NC, N, C), jnp.float32),
    mesh=_vsc_mesh(),
    scratch_types=[
        pltpu.VMEM((EPW,), jnp.int32),
        pltpu.VMEM((EPW,), jnp.int32),
        [pltpu.VMEM((K, C), jnp.float32) for _ in range(NBUF)],
        [pltpu.SemaphoreType.DMA for _ in range(NBUF)],
        pltpu.VMEM_SHARED((N, C), jnp.float32),
    ],
    compiler_params=pltpu.CompilerParams(use_tc_tiling_on_sc=False),
)
def _agg(src_hbm, dst_hbm, h_hbm, out_hbm, sidx, didx, rows, sems, acc):
    cid = lax.axis_index("c")
    sid = lax.axis_index("s")
    wid = cid * NS + sid

    def zrow(i, carry):
        for c16 in range(C // 16):
            rows[0][i, pl.ds(c16 * 16, 16)] = jnp.zeros((16,), jnp.float32)
        return carry
    lax.fori_loop(0, K, zrow, None)
    for t in range(RPT // K):
        pltpu.sync_copy(rows[0], acc.at[pl.ds(sid * RPT + t * K, K)])
    _REM = RPT - (RPT // K) * K
    if _REM:
        pltpu.sync_copy(rows[0].at[pl.ds(0, _REM)],
                        acc.at[pl.ds(sid * RPT + (RPT // K) * K, _REM)])
    plsc.subcore_barrier()

    base = pl.multiple_of(wid * EPW, 8)
    pltpu.sync_copy(src_hbm.at[pl.ds(base, EPW)], sidx)
    pltpu.sync_copy(dst_hbm.at[pl.ds(base, EPW)], didx)

    def _cidx(ref, j):
        return ref.at[pl.ds(pl.multiple_of(j * K, 8), K)]

    def _gather(j, b):
        pltpu.async_copy(h_hbm.at[_cidx(sidx, j)], rows[b], sems[b])

    def _scat(j, b):
        pltpu.make_async_copy(h_hbm.at[pl.ds(0, K)], rows[b], sems[b]).wait()
        pltpu.sync_copy(rows[b], acc.at[_cidx(didx, j)], add=True)

    OUT = NCH // NBUF - 1
    TAIL = NCH - NBUF * OUT

    for b in range(NBUF):
        _gather(b, b)

    def outer(o, carry):
        jb = o * NBUF
        for b in range(NBUF):
            _scat(jb + b, b)
            _gather(jb + b + NBUF, b)
        return carry
    lax.fori_loop(0, OUT, outer, None)

    jb = OUT * NBUF
    for b in range(NBUF):
        _scat(jb + b, b)
    for t in range(TAIL - NBUF):
        j = jb + NBUF + t
        _gather(j, t)
        _scat(j, t)

    plsc.subcore_barrier()
    pltpu.sync_copy(acc.at[pl.ds(sid * RPT, RPT)],
                    out_hbm.at[cid, pl.ds(sid * RPT, RPT)])


BM = 2048


def _norm_cols(degp_ref):
    deg = degp_ref[0] + degp_ref[1]
    nrm = lax.rsqrt(jnp.maximum(deg, 1.0))
    nt = jnp.transpose(nrm, (1, 0))
    return nt[:, 0:1], nt[:, 1:2]


def _prescale_body(x_ref, degp_ref, h_ref):
    ns, _ = _norm_cols(degp_ref)
    h_ref[...] = x_ref[...] * ns


_prescale = pl.pallas_call(
    _prescale_body,
    grid=((N + BM - 1) // BM,),
    in_specs=[
        pl.BlockSpec((BM, C), lambda i: (i, 0)),
        pl.BlockSpec((NC, 2, BM), lambda i: (0, 0, i)),
    ],
    out_specs=pl.BlockSpec((BM, C), lambda i: (i, 0)),
    out_shape=jax.ShapeDtypeStruct((N, C), jnp.float32),
)


def _mm1_body(aggp_ref, degp_ref, w_ref, o_ref):
    ns, nd = _norm_cols(degp_ref)
    a = (aggp_ref[0] + aggp_ref[1]) * nd
    h = jnp.dot(a, w_ref[...], preferred_element_type=jnp.float32)
    o_ref[...] = jnp.maximum(h, 0.0) * ns


_mm1 = pl.pallas_call(
    _mm1_body,
    grid=((N + BM - 1) // BM,),
    in_specs=[
        pl.BlockSpec((NC, BM, C), lambda i: (0, i, 0)),
        pl.BlockSpec((NC, 2, BM), lambda i: (0, 0, i)),
        pl.BlockSpec((C, C), lambda i: (0, 0)),
    ],
    out_specs=pl.BlockSpec((BM, C), lambda i: (i, 0)),
    out_shape=jax.ShapeDtypeStruct((N, C), jnp.float32),
)


def _mm2_body(aggp_ref, degp_ref, w_ref, o_ref):
    _, nd = _norm_cols(degp_ref)
    a = (aggp_ref[0] + aggp_ref[1]) * nd
    o_ref[...] = jnp.dot(a, w_ref[...], preferred_element_type=jnp.float32)


_mm2 = pl.pallas_call(
    _mm2_body,
    grid=((N + BM - 1) // BM,),
    in_specs=[
        pl.BlockSpec((NC, BM, C), lambda i: (0, i, 0)),
        pl.BlockSpec((NC, 2, BM), lambda i: (0, 0, i)),
        pl.BlockSpec((C, C), lambda i: (0, 0)),
    ],
    out_specs=pl.BlockSpec((BM, C), lambda i: (i, 0)),
    out_shape=jax.ShapeDtypeStruct((N, C), jnp.float32),
)


def kernel(x, edge_index, W1, W2):
    ei = edge_index.astype(jnp.int32)
    src = ei[0]
    dst = ei[1]

    degp = _deg(src, dst)
    h = _prescale(x, degp)
    aggp = _agg(src, dst, h)
    h1 = _mm1(aggp, degp, W1)
    aggp2 = _agg(src, dst, h1)
    out = _mm2(aggp2, degp, W2)
    return out


# speedup vs baseline: 2.3501x; 1.0375x over previous
import functools

import jax
import jax.numpy as jnp
from jax import lax
from jax.experimental import pallas as pl
from jax.experimental.pallas import tpu as pltpu
from jax.experimental.pallas import tpu_sc as plsc

N = 10000
E = 320000
C = 128
NC = 2
NS = 16
NW = NC * NS
EPW = E // NW
K = 80
NCH = EPW // K
NBUF = 3
RPT = N // NS


def _vsc_mesh():
    return plsc.VectorSubcoreMesh(core_axis_name="c", subcore_axis_name="s")


@functools.partial(
    pl.kernel,
    out_type=jax.ShapeDtypeStruct((NC, 2, N), jnp.float32),
    mesh=_vsc_mesh(),
    scratch_types=[
        pltpu.VMEM((EPW,), jnp.int32),
        pltpu.VMEM((EPW,), jnp.int32),
        pltpu.VMEM((K,), jnp.float32),
        pltpu.VMEM((2000,), jnp.float32),
        pltpu.VMEM_SHARED((N,), jnp.float32),
        pltpu.VMEM_SHARED((N,), jnp.float32),
        pltpu.SemaphoreType.DMA,
    ],
    compiler_params=pltpu.CompilerParams(use_tc_tiling_on_sc=False),
)
def _deg(ei_hbm, degp_hbm, sidx, didx, ones, zbuf, acc_s, acc_d,
         dsem):
    cid = lax.axis_index("c")
    sid = lax.axis_index("s")
    wid = cid * NS + sid

    @pl.when(sid == 0)
    def _init():
        def zrow(i, carry):
            zbuf[pl.ds(i * 16, 16)] = jnp.zeros((16,), jnp.float32)
            return carry
        lax.fori_loop(0, 2000 // 16, zrow, None)
        for t in range(N // 2000):
            pltpu.sync_copy(zbuf, acc_s.at[pl.ds(t * 2000, 2000)])
            pltpu.sync_copy(zbuf, acc_d.at[pl.ds(t * 2000, 2000)])

    for c16 in range(K // 16):
        ones[pl.ds(c16 * 16, 16)] = jnp.ones((16,), jnp.float32)

    plsc.subcore_barrier()

    base = pl.multiple_of(wid * EPW, 8)
    pltpu.sync_copy(ei_hbm.at[pl.ds(base, EPW)], sidx)
    pltpu.sync_copy(ei_hbm.at[pl.ds(E + base, EPW)], didx)

    DB = 5

    def _cidx(ref, j):
        return ref.at[pl.ds(pl.multiple_of(j * K, 8), K)]

    def _fire(b0):
        for b in range(DB):
            pltpu.async_copy(ones, acc_s.at[_cidx(sidx, b0 + b)], dsem,
                             add=True)
            pltpu.async_copy(ones, acc_d.at[_cidx(didx, b0 + b)], dsem,
                             add=True)

    def _drain():
        for _ in range(2 * DB):
            pltpu.make_async_copy(ones, acc_s.at[_cidx(sidx, 0)],
                                  dsem).wait()

    _fire(0)

    def chunk(o, carry):
        _fire((o + 1) * DB)
        _drain()
        return carry
    lax.fori_loop(0, NCH // DB - 1, chunk, None)
    _drain()

    plsc.subcore_barrier()

    @pl.when(sid == 0)
    def _writeout():
        pltpu.sync_copy(acc_s, degp_hbm.at[cid, 0])
        pltpu.sync_copy(acc_d, degp_hbm.at[cid, 1])


@functools.partial(
    pl.kernel,
    out_type=jax.ShapeDtypeStruct((NC, N, C), jnp.float32),
    mesh=_vsc_mesh(),
    scratch_types=[
        pltpu.VMEM((EPW,), jnp.int32),
        pltpu.VMEM((EPW,), jnp.int32),
        [pltpu.VMEM((K, C), jnp.float32) for _ in range(NBUF)],
        [pltpu.SemaphoreType.DMA for _ in range(NBUF)],
        pltpu.VMEM_SHARED((N, C), jnp.float32),
    ],
    compiler_params=pltpu.CompilerParams(use_tc_tiling_on_sc=False),
)
def _agg(ei_hbm, h_hbm, out_hbm, sidx, didx, rows, sems, acc):
    cid = lax.axis_index("c")
    sid = lax.axis_index("s")
    wid = cid * NS + sid

    def zrow(i, carry):
        for c16 in range(C // 16):
            rows[0][i, pl.ds(c16 * 16, 16)] = jnp.zeros((16,), jnp.float32)
        return carry
    lax.fori_loop(0, K, zrow, None)
    for t in range(RPT // K):
        pltpu.sync_copy(rows[0], acc.at[pl.ds(sid * RPT + t * K, K)])
    _REM = RPT - (RPT // K) * K
    if _REM:
        pltpu.sync_copy(rows[0].at[pl.ds(0, _REM)],
                        acc.at[pl.ds(sid * RPT + (RPT // K) * K, _REM)])
    plsc.subcore_barrier()

    base = pl.multiple_of(wid * EPW, 8)
    pltpu.sync_copy(ei_hbm.at[pl.ds(base, EPW)], sidx)
    pltpu.sync_copy(ei_hbm.at[pl.ds(E + base, EPW)], didx)

    def _cidx(ref, j):
        return ref.at[pl.ds(pl.multiple_of(j * K, 8), K)]

    def _gather(j, b):
        pltpu.async_copy(h_hbm.at[_cidx(sidx, j)], rows[b], sems[b])

    def _scat(j, b):
        pltpu.make_async_copy(h_hbm.at[pl.ds(0, K)], rows[b], sems[b]).wait()
        pltpu.sync_copy(rows[b], acc.at[_cidx(didx, j)], add=True)

    OUT = NCH // NBUF - 1
    TAIL = NCH - NBUF * OUT

    for b in range(NBUF):
        _gather(b, b)

    def outer(o, carry):
        jb = o * NBUF
        for b in range(NBUF):
            _scat(jb + b, b)
            _gather(jb + b + NBUF, b)
        return carry
    lax.fori_loop(0, OUT, outer, None)

    jb = OUT * NBUF
    for b in range(NBUF):
        _scat(jb + b, b)
    for t in range(TAIL - NBUF):
        j = jb + NBUF + t
        _gather(j, t)
        _scat(j, t)

    plsc.subcore_barrier()
    pltpu.sync_copy(acc.at[pl.ds(sid * RPT, RPT)],
                    out_hbm.at[cid, pl.ds(sid * RPT, RPT)])


BM = 2048


def _norm_cols(degp_ref):
    deg = degp_ref[0] + degp_ref[1]
    nrm = lax.rsqrt(jnp.maximum(deg, 1.0))
    nt = jnp.transpose(nrm, (1, 0))
    return nt[:, 0:1], nt[:, 1:2]


def _prescale_body(x_ref, degp_ref, h_ref):
    ns, _ = _norm_cols(degp_ref)
    h_ref[...] = x_ref[...] * ns


_prescale = pl.pallas_call(
    _prescale_body,
    grid=((N + BM - 1) // BM,),
    in_specs=[
        pl.BlockSpec((BM, C), lambda i: (i, 0)),
        pl.BlockSpec((NC, 2, BM), lambda i: (0, 0, i)),
    ],
    out_specs=pl.BlockSpec((BM, C), lambda i: (i, 0)),
    out_shape=jax.ShapeDtypeStruct((N, C), jnp.float32),
)


def _mm1_body(aggp_ref, degp_ref, w_ref, o_ref):
    ns, nd = _norm_cols(degp_ref)
    a = (aggp_ref[0] + aggp_ref[1]) * nd
    h = jnp.dot(a, w_ref[...], preferred_element_type=jnp.float32)
    o_ref[...] = jnp.maximum(h, 0.0) * ns


_mm1 = pl.pallas_call(
    _mm1_body,
    grid=((N + BM - 1) // BM,),
    in_specs=[
        pl.BlockSpec((NC, BM, C), lambda i: (0, i, 0)),
        pl.BlockSpec((NC, 2, BM), lambda i: (0, 0, i)),
        pl.BlockSpec((C, C), lambda i: (0, 0)),
    ],
    out_specs=pl.BlockSpec((BM, C), lambda i: (i, 0)),
    out_shape=jax.ShapeDtypeStruct((N, C), jnp.float32),
)


def _mm2_body(aggp_ref, degp_ref, w_ref, o_ref):
    _, nd = _norm_cols(degp_ref)
    a = (aggp_ref[0] + aggp_ref[1]) * nd
    o_ref[...] = jnp.dot(a, w_ref[...], preferred_element_type=jnp.float32)


_mm2 = pl.pallas_call(
    _mm2_body,
    grid=((N + BM - 1) // BM,),
    in_specs=[
        pl.BlockSpec((NC, BM, C), lambda i: (0, i, 0)),
        pl.BlockSpec((NC, 2, BM), lambda i: (0, 0, i)),
        pl.BlockSpec((C, C), lambda i: (0, 0)),
    ],
    out_specs=pl.BlockSpec((BM, C), lambda i: (i, 0)),
    out_shape=jax.ShapeDtypeStruct((N, C), jnp.float32),
)


def kernel(x, edge_index, W1, W2):
    ei = edge_index.astype(jnp.int32).reshape(2 * E)

    degp = _deg(ei)
    h = _prescale(x, degp)
    aggp = _agg(ei, h)
    h1 = _mm1(aggp, degp, W1)
    aggp2 = _agg(ei, h1)
    out = _mm2(aggp2, degp, W2)
    return out
